# baseline (speedup 1.0000x reference)
import jax
import jax.numpy as jnp
from jax import lax
from jax.experimental import pallas as pl
from jax.experimental.pallas import tpu as pltpu
from jax.experimental.pallas import tpu_sc as plsc

N_NODES = 10000
N_EDGES = 320000
F_IN = 128
F_HID = 256
F_OUT = 128

EB = 128
EP_ROWS = 2560
NC = 2
NS = 16
N_PAD = 10240
IDXC = 16

_MESH = plsc.VectorSubcoreMesh(core_axis_name="c", subcore_axis_name="s")


def _fill_vmem(ref, n_rows, width, value):
    v = jnp.full((16,), value, jnp.float32)

    def body(i, _):
        for j in range(width // 16):
            ref[i, pl.ds(j * 16, 16)] = v
        return 0

    lax.fori_loop(0, n_rows, body, 0)


def _deg_body(zeros_hbm, dst_rs, deg_cat, ones_v, idx_v, acc_sh):
    cid = lax.axis_index("c")
    sid = lax.axis_index("s")
    wid = cid * NS + sid
    rows_per_tile = EP_ROWS // (NC * NS)

    @pl.when(sid == 0)
    def _():
        pltpu.sync_copy(zeros_hbm, acc_sh)

    _fill_vmem(ones_v, EB, EB, 1.0)
    plsc.subcore_barrier()

    def chunk(c, _):
        pltpu.sync_copy(dst_rs.at[pl.ds(wid * rows_per_tile + c * IDXC, IDXC)],
                        idx_v)

        def body(j, _):
            pltpu.sync_copy(ones_v, acc_sh.at[idx_v.at[j]], add=True)
            return 0

        lax.fori_loop(0, IDXC, body, 0)
        return 0

    lax.fori_loop(0, rows_per_tile // IDXC, chunk, 0)
    plsc.subcore_barrier()

    @pl.when(sid == 0)
    def _():
        pltpu.sync_copy(acc_sh, deg_cat.at[pl.ds(cid * N_PAD, N_PAD)])


def _deg_call(zeros_hbm, dst_rs):
    f = pl.kernel(
        _deg_body,
        mesh=_MESH,
        out_type=jax.ShapeDtypeStruct((2 * N_PAD, EB), jnp.float32),
        scratch_types=[
            pltpu.VMEM((EB, EB), jnp.float32),
            pltpu.VMEM((IDXC, EB), jnp.int32),
            pltpu.VMEM_SHARED((N_PAD, EB), jnp.float32),
        ],
    )
    return f(zeros_hbm, dst_rs)


def _agg1_body(zeros_hbm, src_rs3, dst_rs, hs_cat, acc_cat,
               src_v, dst_v, rows_a, rows_b, acc_sh,
               sem_a, sem_b, ssem_a, ssem_b):
    cid = lax.axis_index("c")
    sid = lax.axis_index("s")
    rows_per_tile = EP_ROWS // NS

    @pl.when(sid == 0)
    def _():
        pltpu.sync_copy(zeros_hbm, acc_sh)

    plsc.subcore_barrier()
    rows = (rows_a, rows_b)
    sems = (sem_a, sem_b)
    ssems = (ssem_a, ssem_b)

    def chunk(c, _):
        base = sid * rows_per_tile + c * IDXC
        pltpu.sync_copy(src_rs3.at[cid, pl.ds(base, IDXC)], src_v)
        pltpu.sync_copy(dst_rs.at[pl.ds(base, IDXC)], dst_v)
        cur = pltpu.async_copy(hs_cat.at[src_v.at[0]], rows[0], sems[0])
        sc = [None, None]
        for j in range(IDXC):
            nxt = None
            if j + 1 < IDXC:
                if sc[(j + 1) % 2] is not None:
                    sc[(j + 1) % 2].wait()
                nxt = pltpu.async_copy(hs_cat.at[src_v.at[j + 1]],
                                       rows[(j + 1) % 2], sems[(j + 1) % 2])
            cur.wait()
            sc[j % 2] = pltpu.async_copy(rows[j % 2], acc_sh.at[dst_v.at[j]],
                                         ssems[j % 2], add=True)
            cur = nxt
        for h in sc:
            if h is not None:
                h.wait()
        return 0

    lax.fori_loop(0, rows_per_tile // IDXC, chunk, 0)
    plsc.subcore_barrier()

    @pl.when(sid == 0)
    def _():
        pltpu.sync_copy(acc_sh, acc_cat.at[pl.ds(cid * N_PAD, N_PAD)])


def _agg1_call(zeros_hbm, src_rs3, dst_rs, hs_cat):
    W = F_HID // 2
    f = pl.kernel(
        _agg1_body,
        mesh=_MESH,
        out_type=jax.ShapeDtypeStruct((2 * N_PAD, W), jnp.float32),
        scratch_types=[
            pltpu.VMEM((IDXC, EB), jnp.int32),
            pltpu.VMEM((IDXC, EB), jnp.int32),
            pltpu.VMEM((EB, W), jnp.float32),
            pltpu.VMEM((EB, W), jnp.float32),
            pltpu.VMEM_SHARED((N_PAD, W), jnp.float32),
            pltpu.SemaphoreType.DMA,
            pltpu.SemaphoreType.DMA,
            pltpu.SemaphoreType.DMA,
            pltpu.SemaphoreType.DMA,
        ],
    )
    return f(zeros_hbm, src_rs3, dst_rs, hs_cat)


def _agg2_body(zeros_hbm, src_rs, dst_rs, hs2, acc_cat,
               src_v, dst_v, rows_a, rows_b, acc_sh,
               sem_a, sem_b, ssem_a, ssem_b):
    cid = lax.axis_index("c")
    sid = lax.axis_index("s")
    wid = cid * NS + sid
    rows_per_tile = EP_ROWS // (NC * NS)

    @pl.when(sid == 0)
    def _():
        pltpu.sync_copy(zeros_hbm, acc_sh)

    plsc.subcore_barrier()
    rows = (rows_a, rows_b)
    sems = (sem_a, sem_b)
    ssems = (ssem_a, ssem_b)

    def chunk(c, _):
        base = wid * rows_per_tile + c * IDXC
        pltpu.sync_copy(src_rs.at[pl.ds(base, IDXC)], src_v)
        pltpu.sync_copy(dst_rs.at[pl.ds(base, IDXC)], dst_v)
        cur = pltpu.async_copy(hs2.at[src_v.at[0]], rows[0], sems[0])
        sc = [None, None]
        for j in range(IDXC):
            nxt = None
            if j + 1 < IDXC:
                if sc[(j + 1) % 2] is not None:
                    sc[(j + 1) % 2].wait()
                nxt = pltpu.async_copy(hs2.at[src_v.at[j + 1]],
                                       rows[(j + 1) % 2], sems[(j + 1) % 2])
            cur.wait()
            sc[j % 2] = pltpu.async_copy(rows[j % 2], acc_sh.at[dst_v.at[j]],
                                         ssems[j % 2], add=True)
            cur = nxt
        for h in sc:
            if h is not None:
                h.wait()
        return 0

    lax.fori_loop(0, rows_per_tile // IDXC, chunk, 0)
    plsc.subcore_barrier()

    @pl.when(sid == 0)
    def _():
        pltpu.sync_copy(acc_sh, acc_cat.at[pl.ds(cid * N_PAD, N_PAD)])


def _agg2_call(zeros_hbm, src_rs, dst_rs, hs2):
    f = pl.kernel(
        _agg2_body,
        mesh=_MESH,
        out_type=jax.ShapeDtypeStruct((2 * N_PAD, F_OUT), jnp.float32),
        scratch_types=[
            pltpu.VMEM((IDXC, EB), jnp.int32),
            pltpu.VMEM((IDXC, EB), jnp.int32),
            pltpu.VMEM((EB, F_OUT), jnp.float32),
            pltpu.VMEM((EB, F_OUT), jnp.float32),
            pltpu.VMEM_SHARED((N_PAD, F_OUT), jnp.float32),
            pltpu.SemaphoreType.DMA,
            pltpu.SemaphoreType.DMA,
            pltpu.SemaphoreType.DMA,
            pltpu.SemaphoreType.DMA,
        ],
    )
    return f(zeros_hbm, src_rs, dst_rs, hs2)


_BM = 1024
_GRID = N_PAD // _BM
_OFF = N_PAD // _BM


def _tc1a_body(x_ref, w1_ref, h_ref):
    h_ref[...] = jnp.dot(x_ref[...], w1_ref[...],
                         preferred_element_type=jnp.float32)


def _tc1a_call(x, W1):
    return pl.pallas_call(
        _tc1a_body,
        grid=(_GRID,),
        in_specs=[pl.BlockSpec((_BM, F_IN), lambda i: (i, 0)),
                  pl.BlockSpec((F_IN, F_HID), lambda i: (0, 0))],
        out_specs=pl.BlockSpec((_BM, F_HID), lambda i: (i, 0)),
        out_shape=jax.ShapeDtypeStruct((N_PAD, F_HID), jnp.float32),
    )(x, W1)


def _tc1_body(h_ref, d0_ref, d1_ref, hs_lo_ref, hs_hi_ref, dinv_ref):
    deg = d0_ref[:, :1] + d1_ref[:, :1] + 1.0
    dinv = lax.rsqrt(deg)
    hs = h_ref[...] * dinv
    hs_lo_ref[...] = hs[:, :F_HID // 2]
    hs_hi_ref[...] = hs[:, F_HID // 2:]
    dinv_ref[...] = dinv


def _tc1_call(h1, deg_cat):
    return pl.pallas_call(
        _tc1_body,
        grid=(_GRID,),
        in_specs=[pl.BlockSpec((_BM, F_HID), lambda i: (i, 0)),
                  pl.BlockSpec((_BM, EB), lambda i: (i, 0)),
                  pl.BlockSpec((_BM, EB), lambda i: (i + _OFF, 0))],
        out_specs=[pl.BlockSpec((_BM, F_HID // 2), lambda i: (i, 0)),
                   pl.BlockSpec((_BM, F_HID // 2), lambda i: (i, 0)),
                   pl.BlockSpec((_BM, 1), lambda i: (i, 0))],
        out_shape=[jax.ShapeDtypeStruct((N_PAD, F_HID // 2), jnp.float32),
                   jax.ShapeDtypeStruct((N_PAD, F_HID // 2), jnp.float32),
                   jax.ShapeDtypeStruct((N_PAD, 1), jnp.float32)],
    )(h1, deg_cat, deg_cat)


def _tc2_body(acc_lo_ref, acc_hi_ref, hs_lo_ref, hs_hi_ref, dinv_ref, b1_ref,
              w2_ref, hs2_ref):
    dinv = dinv_ref[...]
    lo = jnp.maximum(dinv * (acc_lo_ref[...] + hs_lo_ref[...])
                     + b1_ref[:, :F_HID // 2], 0.0)
    hi = jnp.maximum(dinv * (acc_hi_ref[...] + hs_hi_ref[...])
                     + b1_ref[:, F_HID // 2:], 0.0)
    out1 = jnp.concatenate([lo, hi], axis=1)
    h2 = jnp.dot(out1, w2_ref[...], preferred_element_type=jnp.float32)
    hs2_ref[...] = h2 * dinv


def _tc2_call(acc_cat, hs_lo, hs_hi, dinv, b1, W2):
    W = F_HID // 2
    return pl.pallas_call(
        _tc2_body,
        grid=(_GRID,),
        in_specs=[pl.BlockSpec((_BM, W), lambda i: (i, 0)),
                  pl.BlockSpec((_BM, W), lambda i: (i + _OFF, 0)),
                  pl.BlockSpec((_BM, W), lambda i: (i, 0)),
                  pl.BlockSpec((_BM, W), lambda i: (i, 0)),
                  pl.BlockSpec((_BM, 1), lambda i: (i, 0)),
                  pl.BlockSpec((1, F_HID), lambda i: (0, 0)),
                  pl.BlockSpec((F_HID, F_OUT), lambda i: (0, 0))],
        out_specs=pl.BlockSpec((_BM, F_OUT), lambda i: (i, 0)),
        out_shape=jax.ShapeDtypeStruct((N_PAD, F_OUT), jnp.float32),
    )(acc_cat, acc_cat, hs_lo, hs_hi, dinv, b1, W2)


def _tc3_body(a_ref, b_ref, hs2_ref, dinv_ref, b2_ref, g_ref, out_ref):
    z = (dinv_ref[...] * (a_ref[...] + b_ref[...] + hs2_ref[...])
         + b2_ref[...] + g_ref[...])
    m = jnp.max(z, axis=1, keepdims=True)
    li = lax.broadcasted_iota(jnp.int32, z.shape, 1)
    first = jnp.min(jnp.where(z >= m, li, z.shape[1]), axis=1, keepdims=True)
    out_ref[...] = (li == first).astype(jnp.float32)


def _tc3_call(acc2_cat, hs2, dinv, b2, g):
    return pl.pallas_call(
        _tc3_body,
        grid=(_GRID,),
        in_specs=[pl.BlockSpec((_BM, F_OUT), lambda i: (i, 0)),
                  pl.BlockSpec((_BM, F_OUT), lambda i: (i + _OFF, 0)),
                  pl.BlockSpec((_BM, F_OUT), lambda i: (i, 0)),
                  pl.BlockSpec((_BM, 1), lambda i: (i, 0)),
                  pl.BlockSpec((1, F_OUT), lambda i: (0, 0)),
                  pl.BlockSpec((_BM, F_OUT), lambda i: (i, 0))],
        out_specs=pl.BlockSpec((_BM, F_OUT), lambda i: (i, 0)),
        out_shape=jax.ShapeDtypeStruct((N_PAD, F_OUT), jnp.float32),
    )(acc2_cat, acc2_cat, hs2, dinv, b2, g)


def _gumbel_const():
    u = jax.random.uniform(jax.random.key(42), (N_NODES, F_OUT), jnp.float32)
    return -jnp.log(-jnp.log(u + 1e-20) + 1e-20)


def kernel(x, edge_index, W1, b1, W2, b2):
    pad = EP_ROWS * EB - N_EDGES
    src_rs = jnp.concatenate(
        [edge_index[0], jnp.zeros((pad,), jnp.int32)]).reshape(EP_ROWS, EB)
    dst_rs = jnp.concatenate(
        [edge_index[1], jnp.full((pad,), N_PAD - 1, jnp.int32)]
    ).reshape(EP_ROWS, EB)
    src_rs3 = jnp.stack([src_rs, src_rs + N_PAD])
    zeros128 = jnp.zeros((N_PAD, EB), jnp.float32)

    deg_cat = _deg_call(zeros128, dst_rs)
    h1 = _tc1a_call(x, W1)
    hs_lo, hs_hi, dinv = _tc1_call(h1, deg_cat)
    hs_cat = jnp.concatenate([hs_lo, hs_hi], axis=0)
    acc_cat = _agg1_call(zeros128, src_rs3, dst_rs, hs_cat)
    hs2 = _tc2_call(acc_cat, hs_lo, hs_hi, dinv, b1.reshape(1, F_HID), W2)
    acc2_cat = _agg2_call(zeros128, src_rs, dst_rs, hs2)
    g = _gumbel_const()
    out = _tc3_call(acc2_cat, hs2, dinv, b2.reshape(1, F_OUT), g)
    return out[:N_NODES]
